# baseline (speedup 1.0000x reference)
import functools
import math

import jax
import jax.numpy as jnp
from jax import lax
from jax.experimental import pallas as pl
from jax.experimental.pallas import tpu as pltpu
from jax.experimental.pallas import tpu_sc as plsc

SEQ_LEN = 2048
BATCH = 64
DIM = 64
VOCAB = 100000
NC = 2
NS = 16
NW = NC * NS
ROWS = SEQ_LEN * BATCH
ROWS_W = ROWS // NW
SEQ_W = SEQ_LEN // NW
CHUNK_S = 2
CHUNK_R = CHUNK_S * BATCH
N_CHUNKS = SEQ_W // CHUNK_S
SCALE = math.sqrt(DIM)
LANES = 16
VPR = DIM // LANES
N_SLOTS = 3
LOOKAHEAD = 2


@functools.cache
def _build_kernel():
    mesh = plsc.VectorSubcoreMesh(
        core_axis_name="c", subcore_axis_name="s", num_cores=NC, num_subcores=NS
    )
    return pl.kernel(
        _emb_body,
        out_type=jax.ShapeDtypeStruct((SEQ_LEN, BATCH, DIM), jnp.float32),
        mesh=mesh,
        scratch_types=[
            pltpu.VMEM((ROWS_W,), jnp.int32),
            pltpu.VMEM((SEQ_W * DIM,), jnp.float32),
            pltpu.VMEM((N_SLOTS, CHUNK_R, 128), jnp.float32),
            pltpu.VMEM((N_SLOTS, CHUNK_S, BATCH, DIM), jnp.float32),
            [pltpu.SemaphoreType.DMA] * N_SLOTS,
            [pltpu.SemaphoreType.DMA] * N_SLOTS,
        ],
    )


def _emb_body(idx_hbm, wt_hbm, pe_hbm, out_hbm, idx_v, pe_v, bufs, obufs, gsems, osems):
    wid = lax.axis_index("s") * NC + lax.axis_index("c")
    base = wid * ROWS_W
    seq_base = wid * SEQ_W

    pltpu.sync_copy(idx_hbm.at[pl.ds(base, ROWS_W)], idx_v)
    pltpu.sync_copy(pe_hbm.at[pl.ds(seq_base * DIM, SEQ_W * DIM)], pe_v)

    def start_gather(g):
        slot = g % N_SLOTS
        idx_slice = idx_v.at[pl.ds(g * CHUNK_R, CHUNK_R)]
        return pltpu.async_copy(wt_hbm.at[idx_slice], bufs.at[slot], gsems[slot])

    def start_out(g):
        slot = g % N_SLOTS
        return [
            pltpu.async_copy(
                obufs.at[slot],
                out_hbm.at[pl.ds(seq_base + g * CHUNK_S, CHUNK_S)],
                osems[slot],
            )
        ]

    gd = {}
    od = {}
    for g in range(LOOKAHEAD):
        gd[g] = start_gather(g)

    for g in range(N_CHUNKS):
        h = g + LOOKAHEAD
        if h < N_CHUNKS:
            prev = h - N_SLOTS
            if prev >= 0:
                for d in od.pop(prev):
                    d.wait()
            gd[h] = start_gather(h)

        gd.pop(g).wait()

        slot = g % N_SLOTS
        for sp in range(CHUNK_S):
            srow = g * CHUNK_S + sp
            pe_regs = [
                pe_v[pl.ds(srow * DIM + j * LANES, LANES)] for j in range(VPR)
            ]

            def row_body(r, c, pe_regs=pe_regs, sp=sp, slot=slot):
                k = sp * BATCH + r
                for j in range(VPR):
                    v = bufs[slot, k, pl.ds(j * LANES, LANES)]
                    obufs[slot, sp, r, pl.ds(j * LANES, LANES)] = v * SCALE + pe_regs[j]
                return c

            lax.fori_loop(0, BATCH, row_body, 0, unroll=2)

        od[g] = start_out(g)

    for g in sorted(od):
        for d in od.pop(g):
            d.wait()


def kernel(source, word_table, pe):
    idx = source.reshape(ROWS)
    wt_wide = jnp.pad(word_table, ((0, 0), (0, 128 - DIM)))
    pe_flat = pe[:SEQ_LEN, 0, :].reshape(SEQ_LEN * DIM)
    return _build_kernel()(idx, wt_wide, pe_flat)
